# Initial kernel scaffold; baseline (speedup 1.0000x reference)
import functools

import jax
import jax.numpy as jnp
from jax import lax
from jax.experimental import pallas as pl
from jax.experimental.pallas import tpu as pltpu
from jax.experimental.pallas import tpu_sc as plsc

VOCAB = 1000000
EMBED_DIM = 32
N_FIELDS = 26
BATCH = 16384
HIDDEN = 256
ALL_DIM = N_FIELDS * EMBED_DIM

NC = 2
NS = 16
NW = NC * NS

TOTAL = BATCH * N_FIELDS
PER_W = TOTAL // NW
CHUNK = 128
CPW = PER_W // CHUNK


def _gather_body(idx_hbm, table_hbm, out_hbm, idx_v, rows_v, sem):
    wid = lax.axis_index("s") * NC + lax.axis_index("c")
    pltpu.sync_copy(idx_hbm.at[wid], idx_v)
    base = wid * PER_W

    def step(j, carry):
        pltpu.async_copy(table_hbm.at[idx_v.at[j]], rows_v, sem).wait()
        pltpu.sync_copy(rows_v, out_hbm.at[pl.ds(base + j * CHUNK, CHUNK)])
        return carry

    lax.fori_loop(0, CPW, step, 0)


_gather = functools.partial(
    pl.kernel,
    out_type=jax.ShapeDtypeStruct((TOTAL, EMBED_DIM), jnp.float32),
    mesh=plsc.VectorSubcoreMesh(core_axis_name="c", subcore_axis_name="s"),
    scratch_types=[
        pltpu.VMEM((CPW, CHUNK), jnp.int32),
        pltpu.VMEM((CHUNK, EMBED_DIM), jnp.float32),
        pltpu.SemaphoreType.DMA,
    ],
)(_gather_body)


def _mlp_body(x_ref, w1_ref, b1_ref, w2_ref, b2_ref, o_ref):
    h = jnp.dot(x_ref[...], w1_ref[...], preferred_element_type=jnp.float32)
    h = jnp.maximum(h + b1_ref[...], 0.0)
    o_ref[...] = (
        jnp.dot(h, w2_ref[...], preferred_element_type=jnp.float32) + b2_ref[...]
    )


MLP_BB = 2048

_mlp = pl.pallas_call(
    _mlp_body,
    grid=(BATCH // MLP_BB,),
    in_specs=[
        pl.BlockSpec((MLP_BB, ALL_DIM), lambda i: (i, 0)),
        pl.BlockSpec((ALL_DIM, HIDDEN), lambda i: (0, 0)),
        pl.BlockSpec((1, HIDDEN), lambda i: (0, 0)),
        pl.BlockSpec((HIDDEN, EMBED_DIM), lambda i: (0, 0)),
        pl.BlockSpec((1, EMBED_DIM), lambda i: (0, 0)),
    ],
    out_specs=pl.BlockSpec((MLP_BB, EMBED_DIM), lambda i: (i, 0)),
    out_shape=jax.ShapeDtypeStruct((BATCH, EMBED_DIM), jnp.float32),
)


def kernel(itemFeatures, table, W1, b1, W2, b2):
    idx3 = itemFeatures.reshape(NW, CPW, CHUNK)
    emb = _gather(idx3, table)
    x = emb.reshape(BATCH, ALL_DIM)
    return _mlp(x, W1, b1.reshape(1, HIDDEN), W2, b2.reshape(1, EMBED_DIM))


# speedup vs baseline: 15.4864x; 15.4864x over previous
import functools

import jax
import jax.numpy as jnp
from jax import lax
from jax.experimental import pallas as pl
from jax.experimental.pallas import tpu as pltpu
from jax.experimental.pallas import tpu_sc as plsc

VOCAB = 1000000
EMBED_DIM = 32
N_FIELDS = 26
BATCH = 16384
HIDDEN = 256
ALL_DIM = N_FIELDS * EMBED_DIM

NC = 2
NS = 16
NW = NC * NS

TOTAL = BATCH * N_FIELDS
PER_W = TOTAL // NW
CHUNK = 128
CPW = PER_W // CHUNK


def _gather_body(idx_hbm, table_hbm, out_hbm, idx_v, rows_v, sem):
    wid = lax.axis_index("s") * NC + lax.axis_index("c")
    pltpu.sync_copy(idx_hbm.at[wid], idx_v)
    base = wid * PER_W

    def step(j, carry):
        pltpu.async_copy(table_hbm.at[idx_v.at[j]], rows_v, sem).wait()
        pltpu.sync_copy(rows_v, out_hbm.at[pl.ds(base + j * CHUNK, CHUNK)])
        return carry

    lax.fori_loop(0, CPW, step, 0)


_gather = functools.partial(
    pl.kernel,
    out_type=jax.ShapeDtypeStruct((TOTAL, EMBED_DIM), jnp.float32),
    mesh=plsc.VectorSubcoreMesh(core_axis_name="c", subcore_axis_name="s"),
    scratch_types=[
        pltpu.VMEM((CPW, CHUNK), jnp.int32),
        pltpu.VMEM((CHUNK, EMBED_DIM), jnp.float32),
        pltpu.SemaphoreType.DMA,
    ],
    compiler_params=pltpu.CompilerParams(use_tc_tiling_on_sc=False),
)(_gather_body)


def _mlp_body(x_ref, w1_ref, b1_ref, w2_ref, b2_ref, o_ref):
    h = jnp.dot(x_ref[...], w1_ref[...], preferred_element_type=jnp.float32)
    h = jnp.maximum(h + b1_ref[...], 0.0)
    o_ref[...] = (
        jnp.dot(h, w2_ref[...], preferred_element_type=jnp.float32) + b2_ref[...]
    )


MLP_BB = 2048

_mlp = pl.pallas_call(
    _mlp_body,
    grid=(BATCH // MLP_BB,),
    in_specs=[
        pl.BlockSpec((MLP_BB, ALL_DIM), lambda i: (i, 0)),
        pl.BlockSpec((ALL_DIM, HIDDEN), lambda i: (0, 0)),
        pl.BlockSpec((1, HIDDEN), lambda i: (0, 0)),
        pl.BlockSpec((HIDDEN, EMBED_DIM), lambda i: (0, 0)),
        pl.BlockSpec((1, EMBED_DIM), lambda i: (0, 0)),
    ],
    out_specs=pl.BlockSpec((MLP_BB, EMBED_DIM), lambda i: (i, 0)),
    out_shape=jax.ShapeDtypeStruct((BATCH, EMBED_DIM), jnp.float32),
)


def kernel(itemFeatures, table, W1, b1, W2, b2):
    idx3 = itemFeatures.reshape(NW, CPW, CHUNK)
    emb = _gather(idx3, table)
    x = emb.reshape(BATCH, ALL_DIM)
    return _mlp(x, W1, b1.reshape(1, HIDDEN), W2, b2.reshape(1, EMBED_DIM))


# speedup vs baseline: 22.4097x; 1.4471x over previous
import functools

import jax
import jax.numpy as jnp
from jax import lax
from jax.experimental import pallas as pl
from jax.experimental.pallas import tpu as pltpu
from jax.experimental.pallas import tpu_sc as plsc

VOCAB = 1000000
EMBED_DIM = 32
N_FIELDS = 26
BATCH = 16384
HIDDEN = 256
ALL_DIM = N_FIELDS * EMBED_DIM

NC = 2
NS = 16
NW = NC * NS

TOTAL = BATCH * N_FIELDS
PER_W = TOTAL // NW
CHUNK = 128
CPW = PER_W // CHUNK


def _gather_body(idx_hbm, table_hbm, out_hbm, idx_v, rows_v, sem):
    wid = lax.axis_index("s") * NC + lax.axis_index("c")
    pltpu.sync_copy(idx_hbm.at[wid], idx_v)
    base = wid * PER_W

    def step(j, carry):
        pltpu.async_copy(table_hbm.at[idx_v.at[j]], rows_v, sem).wait()
        pltpu.sync_copy(rows_v, out_hbm.at[pl.ds(base + j * CHUNK, CHUNK)])
        return carry

    lax.fori_loop(0, CPW, step, 0)


_gather = functools.partial(
    pl.kernel,
    out_type=jax.ShapeDtypeStruct((TOTAL, EMBED_DIM), jnp.float32),
    mesh=plsc.VectorSubcoreMesh(core_axis_name="c", subcore_axis_name="s"),
    scratch_types=[
        pltpu.VMEM((CPW, CHUNK), jnp.int32),
        pltpu.VMEM((CHUNK, EMBED_DIM), jnp.float32),
        pltpu.SemaphoreType.DMA,
    ],
    compiler_params=pltpu.CompilerParams(use_tc_tiling_on_sc=False),
)(_gather_body)


XPB = 2048
XSTEPS = 123
QPAD = XPB * XSTEPS

def _xpose_body(t0, t1, t2, t3, o_ref):
    dn = (((0,), (0,)), ((), ()))
    eye = jnp.eye(EMBED_DIM, dtype=jnp.float32)
    parts = [
        jax.lax.dot_general(t[...], eye, dn, preferred_element_type=jnp.float32)
        for t in (t0, t1, t2, t3)
    ]
    o_ref[...] = jnp.concatenate(parts, axis=1)


_xpose = pl.pallas_call(
    _xpose_body,
    grid=(XSTEPS,),
    in_specs=[
        pl.BlockSpec(
            (EMBED_DIM, XPB),
            lambda i, a=a: (0, jnp.minimum(a * XSTEPS + i, (VOCAB - 1) // XPB)),
        )
        for a in range(4)
    ],
    out_specs=pl.BlockSpec((XPB, 128), lambda i: (i, 0)),
    out_shape=jax.ShapeDtypeStruct((QPAD, 128), jnp.float32),
)


def _mlp_body(x_ref, w1_ref, b1_ref, w2_ref, b2_ref, o_ref):
    h = jnp.dot(x_ref[...], w1_ref[...], preferred_element_type=jnp.float32)
    h = jnp.maximum(h + b1_ref[...], 0.0)
    o_ref[...] = (
        jnp.dot(h, w2_ref[...], preferred_element_type=jnp.float32) + b2_ref[...]
    )


MLP_BB = 2048

_mlp = pl.pallas_call(
    _mlp_body,
    grid=(BATCH // MLP_BB,),
    in_specs=[
        pl.BlockSpec((MLP_BB, ALL_DIM), lambda i: (i, 0)),
        pl.BlockSpec((ALL_DIM, HIDDEN), lambda i: (0, 0)),
        pl.BlockSpec((1, HIDDEN), lambda i: (0, 0)),
        pl.BlockSpec((HIDDEN, EMBED_DIM), lambda i: (0, 0)),
        pl.BlockSpec((1, EMBED_DIM), lambda i: (0, 0)),
    ],
    out_specs=pl.BlockSpec((MLP_BB, EMBED_DIM), lambda i: (i, 0)),
    out_shape=jax.ShapeDtypeStruct((BATCH, EMBED_DIM), jnp.float32),
)


def kernel(itemFeatures, table, W1, b1, W2, b2):
    tT = table.T
    packed = _xpose(tT, tT, tT, tT)
    table_rm = packed.reshape(4 * QPAD, EMBED_DIM)
    idx = 4 * (itemFeatures % QPAD) + itemFeatures // QPAD
    idx3 = idx.reshape(NW, CPW, CHUNK)
    emb = _gather(idx3, table_rm)
    x = emb.reshape(BATCH, ALL_DIM)
    return _mlp(x, W1, b1.reshape(1, HIDDEN), W2, b2.reshape(1, EMBED_DIM))


# speedup vs baseline: 29.3689x; 1.3105x over previous
import functools

import jax
import jax.numpy as jnp
from jax import lax
from jax.experimental import pallas as pl
from jax.experimental.pallas import tpu as pltpu
from jax.experimental.pallas import tpu_sc as plsc

VOCAB = 1000000
EMBED_DIM = 32
N_FIELDS = 26
BATCH = 16384
HIDDEN = 256
ALL_DIM = N_FIELDS * EMBED_DIM

NC = 2
NS = 16
NW = NC * NS

TOTAL = BATCH * N_FIELDS
PER_W = TOTAL // NW
CHUNK = 128
CPW = PER_W // CHUNK


def _gather_body(idx_hbm, table_hbm, out_hbm, idx_v, rows_v, sem):
    wid = lax.axis_index("s") * NC + lax.axis_index("c")
    pltpu.sync_copy(idx_hbm.at[wid], idx_v)
    base = wid * PER_W

    def step(j, carry):
        pltpu.async_copy(table_hbm.at[idx_v.at[j]], rows_v, sem).wait()
        pltpu.sync_copy(rows_v, out_hbm.at[pl.ds(base + j * CHUNK, CHUNK)])
        return carry

    lax.fori_loop(0, CPW, step, 0)


_gather = functools.partial(
    pl.kernel,
    out_type=jax.ShapeDtypeStruct((TOTAL, EMBED_DIM), jnp.float32),
    mesh=plsc.VectorSubcoreMesh(core_axis_name="c", subcore_axis_name="s"),
    scratch_types=[
        pltpu.VMEM((CPW, CHUNK), jnp.int32),
        pltpu.VMEM((CHUNK, EMBED_DIM), jnp.float32),
        pltpu.SemaphoreType.DMA,
    ],
    compiler_params=pltpu.CompilerParams(use_tc_tiling_on_sc=False),
)(_gather_body)


XPB = 2048
XSTEPS = 123
QPAD = XPB * XSTEPS

def _xpose_body(t0, t1, t2, t3, o_ref):
    cat = jnp.concatenate([t0[...], t1[...], t2[...], t3[...]], axis=0)
    o_ref[...] = cat.T


_xpose = pl.pallas_call(
    _xpose_body,
    grid=(XSTEPS,),
    in_specs=[
        pl.BlockSpec(
            (EMBED_DIM, XPB),
            lambda i, a=a: (0, jnp.minimum(a * XSTEPS + i, (VOCAB - 1) // XPB)),
        )
        for a in range(4)
    ],
    out_specs=pl.BlockSpec((XPB, 128), lambda i: (i, 0)),
    out_shape=jax.ShapeDtypeStruct((QPAD, 128), jnp.float32),
)


def _mlp_body(x_ref, w1_ref, b1_ref, w2_ref, b2_ref, o_ref):
    h = jnp.dot(x_ref[...], w1_ref[...], preferred_element_type=jnp.float32)
    h = jnp.maximum(h + b1_ref[...], 0.0)
    o_ref[...] = (
        jnp.dot(h, w2_ref[...], preferred_element_type=jnp.float32) + b2_ref[...]
    )


MLP_BB = 2048

_mlp = pl.pallas_call(
    _mlp_body,
    grid=(BATCH // MLP_BB,),
    in_specs=[
        pl.BlockSpec((MLP_BB, ALL_DIM), lambda i: (i, 0)),
        pl.BlockSpec((ALL_DIM, HIDDEN), lambda i: (0, 0)),
        pl.BlockSpec((1, HIDDEN), lambda i: (0, 0)),
        pl.BlockSpec((HIDDEN, EMBED_DIM), lambda i: (0, 0)),
        pl.BlockSpec((1, EMBED_DIM), lambda i: (0, 0)),
    ],
    out_specs=pl.BlockSpec((MLP_BB, EMBED_DIM), lambda i: (i, 0)),
    out_shape=jax.ShapeDtypeStruct((BATCH, EMBED_DIM), jnp.float32),
)


def kernel(itemFeatures, table, W1, b1, W2, b2):
    tT = table.T
    packed = _xpose(tT, tT, tT, tT)
    table_rm = packed.reshape(4 * QPAD, EMBED_DIM)
    idx = 4 * (itemFeatures % QPAD) + itemFeatures // QPAD
    idx3 = idx.reshape(NW, CPW, CHUNK)
    emb = _gather(idx3, table_rm)
    x = emb.reshape(BATCH, ALL_DIM)
    return _mlp(x, W1, b1.reshape(1, HIDDEN), W2, b2.reshape(1, EMBED_DIM))


# speedup vs baseline: 33.6124x; 1.1445x over previous
import functools

import jax
import jax.numpy as jnp
from jax import lax
from jax.experimental import pallas as pl
from jax.experimental.pallas import tpu as pltpu
from jax.experimental.pallas import tpu_sc as plsc

VOCAB = 1000000
EMBED_DIM = 32
N_FIELDS = 26
BATCH = 16384
HIDDEN = 256
ALL_DIM = N_FIELDS * EMBED_DIM

NC = 2
NS = 16
NW = NC * NS

TOTAL = BATCH * N_FIELDS
PER_W = TOTAL // NW
CHUNK = 128
CPW = PER_W // CHUNK


def _gather_body(idx_hbm, table_hbm, out_hbm, idx_v, rows_v, sem0, sem1):
    wid = lax.axis_index("s") * NC + lax.axis_index("c")
    pltpu.sync_copy(idx_hbm.at[wid], idx_v)
    base = wid * PER_W
    sems = (sem0, sem1)

    pltpu.async_copy(table_hbm.at[idx_v.at[0]], rows_v.at[0], sem0)

    def pair(k, carry):
        j0 = 2 * k
        for p in (0, 1):
            j = j0 + p
            nb = 1 - p

            @pl.when(j + 1 < CPW)
            def _prefetch():
                pltpu.async_copy(
                    table_hbm.at[idx_v.at[j + 1]], rows_v.at[nb], sems[nb]
                )

            pltpu.make_async_copy(
                table_hbm.at[idx_v.at[j]], rows_v.at[p], sems[p]
            ).wait()
            pltpu.sync_copy(
                rows_v.at[p], out_hbm.at[pl.ds(base + j * CHUNK, CHUNK)]
            )
        return carry

    lax.fori_loop(0, CPW // 2, pair, 0)


_gather = functools.partial(
    pl.kernel,
    out_type=jax.ShapeDtypeStruct((TOTAL, EMBED_DIM), jnp.float32),
    mesh=plsc.VectorSubcoreMesh(core_axis_name="c", subcore_axis_name="s"),
    scratch_types=[
        pltpu.VMEM((CPW, CHUNK), jnp.int32),
        pltpu.VMEM((2, CHUNK, EMBED_DIM), jnp.float32),
        pltpu.SemaphoreType.DMA,
        pltpu.SemaphoreType.DMA,
    ],
    compiler_params=pltpu.CompilerParams(use_tc_tiling_on_sc=False),
)(_gather_body)


XPB = 2048
XSTEPS = 123
QPAD = XPB * XSTEPS

def _xpose_body(t0, t1, t2, t3, o_ref):
    cat = jnp.concatenate([t0[...], t1[...], t2[...], t3[...]], axis=0)
    o_ref[...] = cat.T


_xpose = pl.pallas_call(
    _xpose_body,
    grid=(XSTEPS,),
    in_specs=[
        pl.BlockSpec(
            (EMBED_DIM, XPB),
            lambda i, a=a: (0, jnp.minimum(a * XSTEPS + i, (VOCAB - 1) // XPB)),
        )
        for a in range(4)
    ],
    out_specs=pl.BlockSpec((XPB, 128), lambda i: (i, 0)),
    out_shape=jax.ShapeDtypeStruct((QPAD, 128), jnp.float32),
)


def _mlp_body(x_ref, w1_ref, b1_ref, w2_ref, b2_ref, o_ref):
    h = jnp.dot(x_ref[...], w1_ref[...], preferred_element_type=jnp.float32)
    h = jnp.maximum(h + b1_ref[...], 0.0)
    o_ref[...] = (
        jnp.dot(h, w2_ref[...], preferred_element_type=jnp.float32) + b2_ref[...]
    )


MLP_BB = 2048

_mlp = pl.pallas_call(
    _mlp_body,
    grid=(BATCH // MLP_BB,),
    in_specs=[
        pl.BlockSpec((MLP_BB, ALL_DIM), lambda i: (i, 0)),
        pl.BlockSpec((ALL_DIM, HIDDEN), lambda i: (0, 0)),
        pl.BlockSpec((1, HIDDEN), lambda i: (0, 0)),
        pl.BlockSpec((HIDDEN, EMBED_DIM), lambda i: (0, 0)),
        pl.BlockSpec((1, EMBED_DIM), lambda i: (0, 0)),
    ],
    out_specs=pl.BlockSpec((MLP_BB, EMBED_DIM), lambda i: (i, 0)),
    out_shape=jax.ShapeDtypeStruct((BATCH, EMBED_DIM), jnp.float32),
)


def kernel(itemFeatures, table, W1, b1, W2, b2):
    tT = table.T
    packed = _xpose(tT, tT, tT, tT)
    table_rm = packed.reshape(4 * QPAD, EMBED_DIM)
    idx = 4 * (itemFeatures % QPAD) + itemFeatures // QPAD
    idx3 = idx.reshape(NW, CPW, CHUNK)
    emb = _gather(idx3, table_rm)
    x = emb.reshape(BATCH, ALL_DIM)
    return _mlp(x, W1, b1.reshape(1, HIDDEN), W2, b2.reshape(1, EMBED_DIM))


# speedup vs baseline: 41.6588x; 1.2394x over previous
import functools

import jax
import jax.numpy as jnp
import numpy as np
from jax import lax
from jax.experimental import pallas as pl
from jax.experimental.pallas import tpu as pltpu
from jax.experimental.pallas import tpu_sc as plsc

VOCAB = 1000000
EMBED_DIM = 32
N_FIELDS = 26
BATCH = 16384
HIDDEN = 256
ALL_DIM = N_FIELDS * EMBED_DIM

NC = 2
NS = 16
NW = NC * NS

TOTAL = BATCH * N_FIELDS
PER_W = TOTAL // NW
CHUNK = 128
CPW = PER_W // CHUNK

XT = 7
XWORDS = BATCH * XT * 128
XROWS = XWORDS // EMBED_DIM

_n = np.arange(TOTAL)
_b, _f = _n // N_FIELDS, _n % N_FIELDS
DST_ROWS = (
    (_b // 8) * (XT * 32) + (_f // 4) * 32 + (_b % 8) * 4 + (_f % 4)
).astype(np.int32).reshape(NW, CPW, CHUNK)


def _gather_body(idx_hbm, dst_hbm, table_hbm, out_hbm,
                 idx_v, dst_v, rows_v, sem0, sem1, semw):
    wid = lax.axis_index("s") * NC + lax.axis_index("c")
    pltpu.sync_copy(idx_hbm.at[wid], idx_v)
    pltpu.sync_copy(dst_hbm.at[wid], dst_v)
    sems = (sem0, sem1)

    pltpu.async_copy(table_hbm.at[idx_v.at[0]], rows_v.at[0], sem0)

    def pair(k, carry):
        j0 = 2 * k
        for p in (0, 1):
            j = j0 + p
            nb = 1 - p

            @pl.when(j + 1 < CPW)
            def _prefetch():
                pltpu.async_copy(
                    table_hbm.at[idx_v.at[j + 1]], rows_v.at[nb], sems[nb]
                )

            pltpu.make_async_copy(
                table_hbm.at[idx_v.at[j]], rows_v.at[p], sems[p]
            ).wait()
            pltpu.async_copy(
                rows_v.at[p], out_hbm.at[dst_v.at[j]], semw
            ).wait()
        return carry

    lax.fori_loop(0, CPW // 2, pair, 0)


_gather = functools.partial(
    pl.kernel,
    out_type=jax.ShapeDtypeStruct((XROWS, EMBED_DIM), jnp.float32),
    mesh=plsc.VectorSubcoreMesh(core_axis_name="c", subcore_axis_name="s"),
    scratch_types=[
        pltpu.VMEM((CPW, CHUNK), jnp.int32),
        pltpu.VMEM((CPW, CHUNK), jnp.int32),
        pltpu.VMEM((2, CHUNK, EMBED_DIM), jnp.float32),
        pltpu.SemaphoreType.DMA,
        pltpu.SemaphoreType.DMA,
        pltpu.SemaphoreType.DMA,
    ],
    compiler_params=pltpu.CompilerParams(use_tc_tiling_on_sc=False),
)(_gather_body)


XPB = 2048
XSTEPS = 123
QPAD = XPB * XSTEPS


def _xpose_body(t0, t1, t2, t3, o_ref):
    cat = jnp.concatenate([t0[...], t1[...], t2[...], t3[...]], axis=0)
    o_ref[...] = cat.T


_xpose = pl.pallas_call(
    _xpose_body,
    grid=(XSTEPS,),
    in_specs=[
        pl.BlockSpec(
            (EMBED_DIM, XPB),
            lambda i, a=a: (0, jnp.minimum(a * XSTEPS + i, (VOCAB - 1) // XPB)),
        )
        for a in range(4)
    ],
    out_specs=pl.BlockSpec((XPB, 128), lambda i: (i, 0)),
    out_shape=jax.ShapeDtypeStruct((QPAD, 128), jnp.float32),
)


MLP_BB = 2048
MLP_RB = MLP_BB // 8


def _mlp_body(x_ref, w1_ref, b1_ref, w2_ref, b2_ref, o_ref):
    blk = x_ref[...]
    acc = jnp.zeros((MLP_BB, HIDDEN), jnp.float32)
    for g in range(XT):
        xg = blk[:, 8 * g:8 * g + 8, :].reshape(MLP_BB, 128)
        if g == XT - 1:
            lane = lax.broadcasted_iota(jnp.int32, (MLP_BB, 128), 1)
            xg = jnp.where(lane < 64, xg, 0.0)
        acc = acc + jnp.dot(
            xg, w1_ref[g * 128:(g + 1) * 128, :],
            preferred_element_type=jnp.float32,
        )
    h = jnp.maximum(acc + b1_ref[...], 0.0)
    oT = jax.lax.dot_general(
        w2_ref[...], h, (((0,), (1,)), ((), ())),
        preferred_element_type=jnp.float32,
    )
    o_ref[...] = oT + b2_ref[...]


_mlp = pl.pallas_call(
    _mlp_body,
    grid=(BATCH // MLP_BB,),
    in_specs=[
        pl.BlockSpec((MLP_RB, 8 * XT, 128), lambda i: (i, 0, 0)),
        pl.BlockSpec((XT * 128, HIDDEN), lambda i: (0, 0)),
        pl.BlockSpec((1, HIDDEN), lambda i: (0, 0)),
        pl.BlockSpec((HIDDEN, EMBED_DIM), lambda i: (0, 0)),
        pl.BlockSpec((EMBED_DIM, 1), lambda i: (0, 0)),
    ],
    out_specs=pl.BlockSpec((EMBED_DIM, MLP_BB), lambda i: (0, i)),
    out_shape=jax.ShapeDtypeStruct((EMBED_DIM, BATCH), jnp.float32),
)


def kernel(itemFeatures, table, W1, b1, W2, b2):
    tT = table.T
    packed = _xpose(tT, tT, tT, tT)
    table_rm = packed.reshape(4 * QPAD, EMBED_DIM)
    idx = 4 * (itemFeatures % QPAD) + itemFeatures // QPAD
    idx3 = idx.reshape(NW, CPW, CHUNK)
    xt = _gather(idx3, jnp.asarray(DST_ROWS), table_rm)
    x3 = xt.reshape(BATCH // 8, 8 * XT, 128)
    w1p = jnp.pad(W1, ((0, XT * 128 - ALL_DIM), (0, 0)))
    oT = _mlp(x3, w1p, b1.reshape(1, HIDDEN), W2, b2.reshape(EMBED_DIM, 1))
    return oT.T


# speedup vs baseline: 48.5707x; 1.1659x over previous
import functools

import jax
import jax.numpy as jnp
import numpy as np
from jax import lax
from jax.experimental import pallas as pl
from jax.experimental.pallas import tpu as pltpu
from jax.experimental.pallas import tpu_sc as plsc

VOCAB = 1000000
EMBED_DIM = 32
N_FIELDS = 26
BATCH = 16384
HIDDEN = 256
ALL_DIM = N_FIELDS * EMBED_DIM

NC = 2
NS = 16
NW = NC * NS

TOTAL = BATCH * N_FIELDS
PER_W = TOTAL // NW
CHUNK = 128
CPW = PER_W // CHUNK

XT = 7
XWORDS = BATCH * XT * 128
XROWS = XWORDS // EMBED_DIM

_n = np.arange(TOTAL)
_b, _f = _n // N_FIELDS, _n % N_FIELDS
DST_ROWS = (
    (_b // 8) * (XT * 32) + (_f // 4) * 32 + (_b % 8) * 4 + (_f % 4)
).astype(np.int32).reshape(NW, CPW, CHUNK)


def _gather_body(idx_hbm, dst_hbm, table_hbm, out_hbm,
                 idx_v, dst_v, rows_v, sem0, sem1, semw):
    wid = lax.axis_index("s") * NC + lax.axis_index("c")
    pltpu.sync_copy(idx_hbm.at[wid], idx_v)
    pltpu.sync_copy(dst_hbm.at[wid], dst_v)
    sems = (sem0, sem1)

    pltpu.async_copy(table_hbm.at[idx_v.at[0]], rows_v.at[0], sem0)

    def pair(k, carry):
        j0 = 2 * k
        for p in (0, 1):
            j = j0 + p
            nb = 1 - p

            @pl.when(j + 1 < CPW)
            def _prefetch():
                pltpu.async_copy(
                    table_hbm.at[idx_v.at[j + 1]], rows_v.at[nb], sems[nb]
                )

            pltpu.make_async_copy(
                table_hbm.at[idx_v.at[j]], rows_v.at[p], sems[p]
            ).wait()
            pltpu.async_copy(
                rows_v.at[p], out_hbm.at[dst_v.at[j]], semw
            ).wait()
        return carry

    lax.fori_loop(0, CPW // 2, pair, 0)


_gather = functools.partial(
    pl.kernel,
    out_type=jax.ShapeDtypeStruct((XROWS, EMBED_DIM), jnp.float32),
    mesh=plsc.VectorSubcoreMesh(core_axis_name="c", subcore_axis_name="s"),
    scratch_types=[
        pltpu.VMEM((CPW, CHUNK), jnp.int32),
        pltpu.VMEM((CPW, CHUNK), jnp.int32),
        pltpu.VMEM((2, CHUNK, EMBED_DIM), jnp.float32),
        pltpu.SemaphoreType.DMA,
        pltpu.SemaphoreType.DMA,
        pltpu.SemaphoreType.DMA,
    ],
    compiler_params=pltpu.CompilerParams(use_tc_tiling_on_sc=False),
)(_gather_body)


XPB = 4096
XSTEPS = 62
QPAD = XPB * XSTEPS


def _xpose_body(t0, t1, t2, t3, o_ref):
    cat = jnp.concatenate([t0[...], t1[...], t2[...], t3[...]], axis=0)
    o_ref[...] = cat.T


_xpose = pl.pallas_call(
    _xpose_body,
    grid=(XSTEPS,),
    in_specs=[
        pl.BlockSpec(
            (EMBED_DIM, XPB),
            lambda i, a=a: (0, jnp.minimum(a * XSTEPS + i, (VOCAB - 1) // XPB)),
        )
        for a in range(4)
    ],
    out_specs=pl.BlockSpec((XPB, 128), lambda i: (i, 0)),
    out_shape=jax.ShapeDtypeStruct((QPAD, 128), jnp.float32),
)


MLP_BB = 2048
MLP_RB = MLP_BB // 8


def _mlp_body(x_ref, w1_ref, b1_ref, w2_ref, b2_ref, o_ref):
    blk = x_ref[...]
    acc = jnp.zeros((MLP_BB, HIDDEN), jnp.float32)
    for g in range(XT):
        xg = blk[:, 8 * g:8 * g + 8, :].reshape(MLP_BB, 128)
        if g == XT - 1:
            lane = lax.broadcasted_iota(jnp.int32, (MLP_BB, 128), 1)
            xg = jnp.where(lane < 64, xg, 0.0)
        acc = acc + jnp.dot(
            xg, w1_ref[g * 128:(g + 1) * 128, :],
            preferred_element_type=jnp.float32,
        )
    h = jnp.maximum(acc + b1_ref[...], 0.0)
    oT = jax.lax.dot_general(
        w2_ref[...], h, (((0,), (1,)), ((), ())),
        preferred_element_type=jnp.float32,
    )
    o_ref[...] = oT + b2_ref[...]


_mlp = pl.pallas_call(
    _mlp_body,
    grid=(BATCH // MLP_BB,),
    in_specs=[
        pl.BlockSpec((MLP_RB, 8 * XT, 128), lambda i: (i, 0, 0)),
        pl.BlockSpec((XT * 128, HIDDEN), lambda i: (0, 0)),
        pl.BlockSpec((1, HIDDEN), lambda i: (0, 0)),
        pl.BlockSpec((HIDDEN, EMBED_DIM), lambda i: (0, 0)),
        pl.BlockSpec((EMBED_DIM, 1), lambda i: (0, 0)),
    ],
    out_specs=pl.BlockSpec((EMBED_DIM, MLP_BB), lambda i: (0, i)),
    out_shape=jax.ShapeDtypeStruct((EMBED_DIM, BATCH), jnp.float32),
)


def kernel(itemFeatures, table, W1, b1, W2, b2):
    tT = table.T
    packed = _xpose(tT, tT, tT, tT)
    table_rm = packed.reshape(4 * QPAD, EMBED_DIM)
    idx = 4 * (itemFeatures % QPAD) + itemFeatures // QPAD
    idx3 = idx.reshape(NW, CPW, CHUNK)
    xt = _gather(idx3, jnp.asarray(DST_ROWS), table_rm)
    x3 = xt.reshape(BATCH // 8, 8 * XT, 128)
    w1p = jnp.pad(W1, ((0, XT * 128 - ALL_DIM), (0, 0)))
    oT = _mlp(x3, w1p, b1.reshape(1, HIDDEN), W2, b2.reshape(EMBED_DIM, 1))
    return oT.T


# speedup vs baseline: 51.7057x; 1.0645x over previous
import functools

import jax
import jax.numpy as jnp
import numpy as np
from jax import lax
from jax.experimental import pallas as pl
from jax.experimental.pallas import tpu as pltpu
from jax.experimental.pallas import tpu_sc as plsc

VOCAB = 1000000
EMBED_DIM = 32
N_FIELDS = 26
BATCH = 16384
HIDDEN = 256
ALL_DIM = N_FIELDS * EMBED_DIM

NC = 2
NS = 16
NW = NC * NS

TOTAL = BATCH * N_FIELDS
PER_W = TOTAL // NW
CHUNK = 128
CPW = PER_W // CHUNK

XT = 7
XWORDS = BATCH * XT * 128
XROWS = XWORDS // EMBED_DIM

_n = np.arange(TOTAL)
_b, _f = _n // N_FIELDS, _n % N_FIELDS
DST_ROWS = (
    (_b // 8) * (XT * 32) + (_f // 4) * 32 + (_b % 8) * 4 + (_f % 4)
).astype(np.int32).reshape(NW, CPW, CHUNK)


def _gather_body(idx_hbm, dst_hbm, table_hbm, out_hbm,
                 idx_v, dst_v, rows_v, sem0, sem1, semw):
    wid = lax.axis_index("s") * NC + lax.axis_index("c")
    pltpu.sync_copy(idx_hbm.at[wid], idx_v)
    pltpu.sync_copy(dst_hbm.at[wid], dst_v)
    sems = (sem0, sem1)

    pltpu.async_copy(table_hbm.at[idx_v.at[0]], rows_v.at[0], sem0)

    def pair(k, carry):
        j0 = 2 * k
        for p in (0, 1):
            j = j0 + p
            nb = 1 - p

            @pl.when(j + 1 < CPW)
            def _prefetch():
                pltpu.async_copy(
                    table_hbm.at[idx_v.at[j + 1]], rows_v.at[nb], sems[nb]
                )

            pltpu.make_async_copy(
                table_hbm.at[idx_v.at[j]], rows_v.at[p], sems[p]
            ).wait()
            pltpu.async_copy(
                rows_v.at[p], out_hbm.at[dst_v.at[j]], semw
            ).wait()
        return carry

    lax.fori_loop(0, CPW // 2, pair, 0)


_gather = functools.partial(
    pl.kernel,
    out_type=jax.ShapeDtypeStruct((XROWS, EMBED_DIM), jnp.float32),
    mesh=plsc.VectorSubcoreMesh(core_axis_name="c", subcore_axis_name="s"),
    scratch_types=[
        pltpu.VMEM((CPW, CHUNK), jnp.int32),
        pltpu.VMEM((CPW, CHUNK), jnp.int32),
        pltpu.VMEM((2, CHUNK, EMBED_DIM), jnp.float32),
        pltpu.SemaphoreType.DMA,
        pltpu.SemaphoreType.DMA,
        pltpu.SemaphoreType.DMA,
    ],
    compiler_params=pltpu.CompilerParams(use_tc_tiling_on_sc=False),
)(_gather_body)


XPB = 8192
XSTEPS = 31
QPAD = XPB * XSTEPS


def _xpose_body(t0, t1, t2, t3, o_ref):
    cat = jnp.concatenate([t0[...], t1[...], t2[...], t3[...]], axis=0)
    o_ref[...] = cat.T


_xpose = pl.pallas_call(
    _xpose_body,
    grid=(XSTEPS,),
    in_specs=[
        pl.BlockSpec(
            (EMBED_DIM, XPB),
            lambda i, a=a: (0, jnp.minimum(a * XSTEPS + i, (VOCAB - 1) // XPB)),
        )
        for a in range(4)
    ],
    out_specs=pl.BlockSpec((XPB, 128), lambda i: (i, 0)),
    out_shape=jax.ShapeDtypeStruct((QPAD, 128), jnp.float32),
)


MLP_BB = 2048
MLP_RB = MLP_BB // 8


def _mlp_body(x_ref, w1_ref, b1_ref, w2_ref, b2_ref, o_ref):
    blk = x_ref[...]
    acc = jnp.zeros((MLP_BB, HIDDEN), jnp.float32)
    for g in range(XT):
        xg = blk[:, 8 * g:8 * g + 8, :].reshape(MLP_BB, 128)
        if g == XT - 1:
            lane = lax.broadcasted_iota(jnp.int32, (MLP_BB, 128), 1)
            xg = jnp.where(lane < 64, xg, 0.0)
        acc = acc + jnp.dot(
            xg, w1_ref[g * 128:(g + 1) * 128, :],
            preferred_element_type=jnp.float32,
        )
    h = jnp.maximum(acc + b1_ref[...], 0.0)
    oT = jax.lax.dot_general(
        w2_ref[...], h, (((0,), (1,)), ((), ())),
        preferred_element_type=jnp.float32,
    )
    o_ref[...] = oT + b2_ref[...]


_mlp = pl.pallas_call(
    _mlp_body,
    grid=(BATCH // MLP_BB,),
    in_specs=[
        pl.BlockSpec((MLP_RB, 8 * XT, 128), lambda i: (i, 0, 0)),
        pl.BlockSpec((XT * 128, HIDDEN), lambda i: (0, 0)),
        pl.BlockSpec((1, HIDDEN), lambda i: (0, 0)),
        pl.BlockSpec((HIDDEN, EMBED_DIM), lambda i: (0, 0)),
        pl.BlockSpec((EMBED_DIM, 1), lambda i: (0, 0)),
    ],
    out_specs=pl.BlockSpec((EMBED_DIM, MLP_BB), lambda i: (0, i)),
    out_shape=jax.ShapeDtypeStruct((EMBED_DIM, BATCH), jnp.float32),
)


def kernel(itemFeatures, table, W1, b1, W2, b2):
    tT = table.T
    packed = _xpose(tT, tT, tT, tT)
    table_rm = packed.reshape(4 * QPAD, EMBED_DIM)
    idx = 4 * (itemFeatures % QPAD) + itemFeatures // QPAD
    idx3 = idx.reshape(NW, CPW, CHUNK)
    xt = _gather(idx3, jnp.asarray(DST_ROWS), table_rm)
    x3 = xt.reshape(BATCH // 8, 8 * XT, 128)
    w1p = jnp.pad(W1, ((0, XT * 128 - ALL_DIM), (0, 0)))
    oT = _mlp(x3, w1p, b1.reshape(1, HIDDEN), W2, b2.reshape(EMBED_DIM, 1))
    return oT.T


# speedup vs baseline: 52.0999x; 1.0076x over previous
import functools

import jax
import jax.numpy as jnp
import numpy as np
from jax import lax
from jax.experimental import pallas as pl
from jax.experimental.pallas import tpu as pltpu
from jax.experimental.pallas import tpu_sc as plsc

VOCAB = 1000000
EMBED_DIM = 32
N_FIELDS = 26
BATCH = 16384
HIDDEN = 256
ALL_DIM = N_FIELDS * EMBED_DIM

NC = 2
NS = 16
NW = NC * NS

TOTAL = BATCH * N_FIELDS
PER_W = TOTAL // NW
CHUNK = 128
CPW = PER_W // CHUNK

XT = 7
XWORDS = BATCH * XT * 128
XROWS = XWORDS // EMBED_DIM

_n = np.arange(TOTAL)
_b, _f = _n // N_FIELDS, _n % N_FIELDS
DST_ROWS = (
    (_b // 8) * (XT * 32) + (_f // 4) * 32 + (_b % 8) * 4 + (_f % 4)
).astype(np.int32).reshape(NW, CPW, CHUNK)


def _gather_body(idx_hbm, dst_hbm, table_hbm, out_hbm,
                 idx_v, dst_v, rows_v, sem0, sem1, semw):
    wid = lax.axis_index("s") * NC + lax.axis_index("c")
    pltpu.sync_copy(idx_hbm.at[wid], idx_v)
    pltpu.sync_copy(dst_hbm.at[wid], dst_v)
    sems = (sem0, sem1)

    pltpu.async_copy(table_hbm.at[idx_v.at[0]], rows_v.at[0], sem0)

    def pair(k, carry):
        j0 = 2 * k
        for p in (0, 1):
            j = j0 + p
            nb = 1 - p

            @pl.when(j + 1 < CPW)
            def _prefetch():
                pltpu.async_copy(
                    table_hbm.at[idx_v.at[j + 1]], rows_v.at[nb], sems[nb]
                )

            pltpu.make_async_copy(
                table_hbm.at[idx_v.at[j]], rows_v.at[p], sems[p]
            ).wait()
            pltpu.async_copy(
                rows_v.at[p], out_hbm.at[dst_v.at[j]], semw
            ).wait()
        return carry

    lax.fori_loop(0, CPW // 2, pair, 0)


_gather = functools.partial(
    pl.kernel,
    out_type=jax.ShapeDtypeStruct((XROWS, EMBED_DIM), jnp.float32),
    mesh=plsc.VectorSubcoreMesh(core_axis_name="c", subcore_axis_name="s"),
    scratch_types=[
        pltpu.VMEM((CPW, CHUNK), jnp.int32),
        pltpu.VMEM((CPW, CHUNK), jnp.int32),
        pltpu.VMEM((2, CHUNK, EMBED_DIM), jnp.float32),
        pltpu.SemaphoreType.DMA,
        pltpu.SemaphoreType.DMA,
        pltpu.SemaphoreType.DMA,
    ],
    compiler_params=pltpu.CompilerParams(use_tc_tiling_on_sc=False),
)(_gather_body)


XPB = 16384
XSTEPS = 16
QPAD = XPB * XSTEPS


def _xpose_body(t0, t1, t2, t3, o_ref):
    cat = jnp.concatenate([t0[...], t1[...], t2[...], t3[...]], axis=0)
    o_ref[...] = cat.T


_xpose = pl.pallas_call(
    _xpose_body,
    grid=(XSTEPS,),
    in_specs=[
        pl.BlockSpec(
            (EMBED_DIM, XPB),
            lambda i, a=a: (0, jnp.minimum(a * XSTEPS + i, (VOCAB - 1) // XPB)),
        )
        for a in range(4)
    ],
    out_specs=pl.BlockSpec((XPB, 128), lambda i: (i, 0)),
    out_shape=jax.ShapeDtypeStruct((QPAD, 128), jnp.float32),
)


MLP_BB = 2048
MLP_RB = MLP_BB // 8


def _mlp_body(x_ref, w1_ref, b1_ref, w2_ref, b2_ref, o_ref):
    blk = x_ref[...]
    acc = jnp.zeros((MLP_BB, HIDDEN), jnp.float32)
    for g in range(XT):
        xg = blk[:, 8 * g:8 * g + 8, :].reshape(MLP_BB, 128)
        if g == XT - 1:
            lane = lax.broadcasted_iota(jnp.int32, (MLP_BB, 128), 1)
            xg = jnp.where(lane < 64, xg, 0.0)
        acc = acc + jnp.dot(
            xg, w1_ref[g * 128:(g + 1) * 128, :],
            preferred_element_type=jnp.float32,
        )
    h = jnp.maximum(acc + b1_ref[...], 0.0)
    oT = jax.lax.dot_general(
        w2_ref[...], h, (((0,), (1,)), ((), ())),
        preferred_element_type=jnp.float32,
    )
    o_ref[...] = oT + b2_ref[...]


_mlp = pl.pallas_call(
    _mlp_body,
    grid=(BATCH // MLP_BB,),
    in_specs=[
        pl.BlockSpec((MLP_RB, 8 * XT, 128), lambda i: (i, 0, 0)),
        pl.BlockSpec((XT * 128, HIDDEN), lambda i: (0, 0)),
        pl.BlockSpec((1, HIDDEN), lambda i: (0, 0)),
        pl.BlockSpec((HIDDEN, EMBED_DIM), lambda i: (0, 0)),
        pl.BlockSpec((EMBED_DIM, 1), lambda i: (0, 0)),
    ],
    out_specs=pl.BlockSpec((EMBED_DIM, MLP_BB), lambda i: (0, i)),
    out_shape=jax.ShapeDtypeStruct((EMBED_DIM, BATCH), jnp.float32),
)


def kernel(itemFeatures, table, W1, b1, W2, b2):
    tT = table.T
    packed = _xpose(tT, tT, tT, tT)
    table_rm = packed.reshape(4 * QPAD, EMBED_DIM)
    idx = 4 * (itemFeatures % QPAD) + itemFeatures // QPAD
    idx3 = idx.reshape(NW, CPW, CHUNK)
    xt = _gather(idx3, jnp.asarray(DST_ROWS), table_rm)
    x3 = xt.reshape(BATCH // 8, 8 * XT, 128)
    w1p = jnp.pad(W1, ((0, XT * 128 - ALL_DIM), (0, 0)))
    oT = _mlp(x3, w1p, b1.reshape(1, HIDDEN), W2, b2.reshape(EMBED_DIM, 1))
    return oT.T
